# Initial kernel scaffold; baseline (speedup 1.0000x reference)
import jax
import jax.numpy as jnp
from jax.experimental import pallas as pl


def kernel(pids, cond_feat, ctx, W1, b1, W2, b2, mask_token):
    raise NotImplementedError("write your pallas kernel here")


# speedup vs baseline: 7.2523x; 7.2523x over previous
import functools

import jax
import jax.numpy as jnp
from jax.experimental import pallas as pl

B = 4096
N_CTX = 16
CTX_DIM = 512
COND_DIM = 512
K = 8
BB = 256


def _fused_body(cond_ref, ctx_ref, W1_ref, b1_ref, W2_ref, b2_ref, mt_ref,
                cf_ref, cm_ref, tf_ref, tm_ref):
    cond = cond_ref[...]
    w1 = W1_ref[...]
    w2 = W2_ref[...]
    h = jax.lax.dot_general(cond, w1, (((1,), (1,)), ((), ())),
                            preferred_element_type=jnp.float32)
    h = jnp.maximum(h + b1_ref[...], 0.0)
    bias = jax.lax.dot_general(h, w2, (((1,), (1,)), ((), ())),
                               preferred_element_type=jnp.float32)
    bias = bias + b2_ref[...]

    cf = ctx_ref[...][None, :, :] + bias[:, None, :]
    cf_ref[...] = cf

    n2 = jnp.sum(cf * cf, axis=-1)
    num = jnp.sum(cf * cond[:, None, :], axis=-1)
    s = num / jnp.maximum(jnp.sqrt(n2), 1e-6)

    st = s.T
    n_io = jax.lax.broadcasted_iota(jnp.int32, (N_CTX, BB), 0)
    rank_t = jnp.zeros((N_CTX, BB), jnp.float32)
    for m in range(N_CTX):
        smrow = st[m:m + 1, :]
        beats_m = (smrow > st) | ((smrow == st) & (m < n_io))
        rank_t = rank_t + beats_m.astype(jnp.float32)
    rank = rank_t.T
    masked = jax.lax.broadcast_in_dim(rank, (BB, N_CTX, 1), (0, 1)) < float(K)

    cm = jnp.where(masked, mt_ref[...][None, :, :], cf)
    cm_ref[...] = cm
    tf_ref[...] = jnp.mean(cf, axis=1)
    tm_ref[...] = jnp.mean(cm, axis=1)


@functools.partial(jax.jit, static_argnames=())
def kernel(pids, cond_feat, ctx, W1, b1, W2, b2, mask_token):
    del pids
    b1r = b1.reshape(1, CTX_DIM)
    b2r = b2.reshape(1, CTX_DIM)
    mtr = mask_token.reshape(1, CTX_DIM)

    grid = (B // BB,)
    const2 = lambda i: (0, 0)
    out = pl.pallas_call(
        _fused_body,
        grid=grid,
        in_specs=[
            pl.BlockSpec((BB, COND_DIM), lambda i: (i, 0)),
            pl.BlockSpec((N_CTX, CTX_DIM), const2),
            pl.BlockSpec((CTX_DIM, COND_DIM), const2),
            pl.BlockSpec((1, CTX_DIM), const2),
            pl.BlockSpec((CTX_DIM, CTX_DIM), const2),
            pl.BlockSpec((1, CTX_DIM), const2),
            pl.BlockSpec((1, CTX_DIM), const2),
        ],
        out_specs=[
            pl.BlockSpec((BB, N_CTX, CTX_DIM), lambda i: (i, 0, 0)),
            pl.BlockSpec((BB, N_CTX, CTX_DIM), lambda i: (i, 0, 0)),
            pl.BlockSpec((BB, CTX_DIM), lambda i: (i, 0)),
            pl.BlockSpec((BB, CTX_DIM), lambda i: (i, 0)),
        ],
        out_shape=[
            jax.ShapeDtypeStruct((B, N_CTX, CTX_DIM), jnp.float32),
            jax.ShapeDtypeStruct((B, N_CTX, CTX_DIM), jnp.float32),
            jax.ShapeDtypeStruct((B, CTX_DIM), jnp.float32),
            jax.ShapeDtypeStruct((B, CTX_DIM), jnp.float32),
        ],
    )(cond_feat, ctx, W1, b1r, W2, b2r, mtr)
    ctx_full, ctx_masked, txt_full, txt_masked = out
    return (ctx_full, ctx_masked, txt_full, txt_masked)


# speedup vs baseline: 8.7133x; 1.2014x over previous
import functools

import jax
import jax.numpy as jnp
from jax.experimental import pallas as pl

B = 4096
N_CTX = 16
CTX_DIM = 512
COND_DIM = 512
K = 8
BB = 256


def _fused_body(cond_ref, ctx_ref, W1_ref, b1_ref, W2_ref, b2_ref, mt_ref,
                cf_ref, cm_ref, tf_ref, tm_ref):
    cond = cond_ref[...]
    w1 = W1_ref[...]
    w2 = W2_ref[...]
    h = jax.lax.dot_general(cond, w1, (((1,), (1,)), ((), ())),
                            preferred_element_type=jnp.float32)
    h = jnp.maximum(h + b1_ref[...], 0.0)
    bias = jax.lax.dot_general(h, w2, (((1,), (1,)), ((), ())),
                               preferred_element_type=jnp.float32)
    bias = bias + b2_ref[...]

    cf = ctx_ref[...][None, :, :] + bias[:, None, :]
    cf_ref[...] = cf

    n2 = jnp.sum(cf * cf, axis=-1)
    num = jnp.sum(cf * cond[:, None, :], axis=-1)
    s = num / jnp.maximum(jnp.sqrt(n2), 1e-6)

    st = s.T
    n_io = jax.lax.broadcasted_iota(jnp.int32, (N_CTX, BB), 0)
    rank_t = jnp.zeros((N_CTX, BB), jnp.float32)
    for m in range(N_CTX):
        smrow = st[m:m + 1, :]
        beats_m = (smrow > st) | ((smrow == st) & (m < n_io))
        rank_t = rank_t + beats_m.astype(jnp.float32)
    rank = rank_t.T
    masked = jax.lax.broadcast_in_dim(rank, (BB, N_CTX, 1), (0, 1)) < float(K)

    mt = mt_ref[...]
    cm_ref[...] = jnp.where(masked, mt[None, :, :], cf)

    ctx_sum = jnp.sum(ctx_ref[...], axis=0, keepdims=True)
    tf_ref[...] = (1.0 / N_CTX) * ctx_sum + bias
    keep = (rank >= float(K)).astype(jnp.float32)
    kctx = jax.lax.dot_general(keep, ctx_ref[...], (((1,), (0,)), ((), ())),
                               preferred_element_type=jnp.float32)
    tm_ref[...] = (1.0 / N_CTX) * (kctx + float(N_CTX - K) * bias
                                   + float(K) * mt)


@functools.partial(jax.jit, static_argnames=())
def kernel(pids, cond_feat, ctx, W1, b1, W2, b2, mask_token):
    del pids
    b1r = b1.reshape(1, CTX_DIM)
    b2r = b2.reshape(1, CTX_DIM)
    mtr = mask_token.reshape(1, CTX_DIM)

    grid = (B // BB,)
    const2 = lambda i: (0, 0)
    out = pl.pallas_call(
        _fused_body,
        grid=grid,
        in_specs=[
            pl.BlockSpec((BB, COND_DIM), lambda i: (i, 0)),
            pl.BlockSpec((N_CTX, CTX_DIM), const2),
            pl.BlockSpec((CTX_DIM, COND_DIM), const2),
            pl.BlockSpec((1, CTX_DIM), const2),
            pl.BlockSpec((CTX_DIM, CTX_DIM), const2),
            pl.BlockSpec((1, CTX_DIM), const2),
            pl.BlockSpec((1, CTX_DIM), const2),
        ],
        out_specs=[
            pl.BlockSpec((BB, N_CTX, CTX_DIM), lambda i: (i, 0, 0)),
            pl.BlockSpec((BB, N_CTX, CTX_DIM), lambda i: (i, 0, 0)),
            pl.BlockSpec((BB, CTX_DIM), lambda i: (i, 0)),
            pl.BlockSpec((BB, CTX_DIM), lambda i: (i, 0)),
        ],
        out_shape=[
            jax.ShapeDtypeStruct((B, N_CTX, CTX_DIM), jnp.float32),
            jax.ShapeDtypeStruct((B, N_CTX, CTX_DIM), jnp.float32),
            jax.ShapeDtypeStruct((B, CTX_DIM), jnp.float32),
            jax.ShapeDtypeStruct((B, CTX_DIM), jnp.float32),
        ],
    )(cond_feat, ctx, W1, b1r, W2, b2r, mtr)
    ctx_full, ctx_masked, txt_full, txt_masked = out
    return (ctx_full, ctx_masked, txt_full, txt_masked)


# speedup vs baseline: 8.7427x; 1.0034x over previous
import functools

import jax
import jax.numpy as jnp
from jax.experimental import pallas as pl

B = 4096
N_CTX = 16
CTX_DIM = 512
COND_DIM = 512
K = 8
BB = 128


def _fused_body(cond_ref, ctx_ref, W1_ref, b1_ref, W2_ref, b2_ref, mt_ref,
                cf_ref, cm_ref, tf_ref, tm_ref):
    cond = cond_ref[...]
    w1 = W1_ref[...]
    w2 = W2_ref[...]
    h = jax.lax.dot_general(cond, w1, (((1,), (1,)), ((), ())),
                            preferred_element_type=jnp.float32)
    h = jnp.maximum(h + b1_ref[...], 0.0)
    bias = jax.lax.dot_general(h, w2, (((1,), (1,)), ((), ())),
                               preferred_element_type=jnp.float32)
    bias = bias + b2_ref[...]

    cf = ctx_ref[...][None, :, :] + bias[:, None, :]
    cf_ref[...] = cf

    n2 = jnp.sum(cf * cf, axis=-1)
    num = jnp.sum(cf * cond[:, None, :], axis=-1)
    s = num / jnp.maximum(jnp.sqrt(n2), 1e-6)

    st = s.T
    n_io = jax.lax.broadcasted_iota(jnp.int32, (N_CTX, BB), 0)
    rank_t = jnp.zeros((N_CTX, BB), jnp.float32)
    for m in range(N_CTX):
        smrow = st[m:m + 1, :]
        beats_m = (smrow > st) | ((smrow == st) & (m < n_io))
        rank_t = rank_t + beats_m.astype(jnp.float32)
    rank = rank_t.T
    masked = jax.lax.broadcast_in_dim(rank, (BB, N_CTX, 1), (0, 1)) < float(K)

    mt = mt_ref[...]
    cm_ref[...] = jnp.where(masked, mt[None, :, :], cf)

    ctx_sum = jnp.sum(ctx_ref[...], axis=0, keepdims=True)
    tf_ref[...] = (1.0 / N_CTX) * ctx_sum + bias
    keep = (rank >= float(K)).astype(jnp.float32)
    kctx = jax.lax.dot_general(keep, ctx_ref[...], (((1,), (0,)), ((), ())),
                               preferred_element_type=jnp.float32)
    tm_ref[...] = (1.0 / N_CTX) * (kctx + float(N_CTX - K) * bias
                                   + float(K) * mt)


@functools.partial(jax.jit, static_argnames=())
def kernel(pids, cond_feat, ctx, W1, b1, W2, b2, mask_token):
    del pids
    b1r = b1.reshape(1, CTX_DIM)
    b2r = b2.reshape(1, CTX_DIM)
    mtr = mask_token.reshape(1, CTX_DIM)

    grid = (B // BB,)
    const2 = lambda i: (0, 0)
    out = pl.pallas_call(
        _fused_body,
        grid=grid,
        in_specs=[
            pl.BlockSpec((BB, COND_DIM), lambda i: (i, 0)),
            pl.BlockSpec((N_CTX, CTX_DIM), const2),
            pl.BlockSpec((CTX_DIM, COND_DIM), const2),
            pl.BlockSpec((1, CTX_DIM), const2),
            pl.BlockSpec((CTX_DIM, CTX_DIM), const2),
            pl.BlockSpec((1, CTX_DIM), const2),
            pl.BlockSpec((1, CTX_DIM), const2),
        ],
        out_specs=[
            pl.BlockSpec((BB, N_CTX, CTX_DIM), lambda i: (i, 0, 0)),
            pl.BlockSpec((BB, N_CTX, CTX_DIM), lambda i: (i, 0, 0)),
            pl.BlockSpec((BB, CTX_DIM), lambda i: (i, 0)),
            pl.BlockSpec((BB, CTX_DIM), lambda i: (i, 0)),
        ],
        out_shape=[
            jax.ShapeDtypeStruct((B, N_CTX, CTX_DIM), jnp.float32),
            jax.ShapeDtypeStruct((B, N_CTX, CTX_DIM), jnp.float32),
            jax.ShapeDtypeStruct((B, CTX_DIM), jnp.float32),
            jax.ShapeDtypeStruct((B, CTX_DIM), jnp.float32),
        ],
    )(cond_feat, ctx, W1, b1r, W2, b2r, mtr)
    ctx_full, ctx_masked, txt_full, txt_masked = out
    return (ctx_full, ctx_masked, txt_full, txt_masked)


# speedup vs baseline: 8.8420x; 1.0114x over previous
import functools

import jax
import jax.numpy as jnp
from jax.experimental import pallas as pl

B = 4096
N_CTX = 16
CTX_DIM = 512
COND_DIM = 512
K = 8
BB = 256


def _fused_body(cond_ref, ctx_ref, W1_ref, b1_ref, W2_ref, b2_ref, mt_ref,
                cf_ref, cm_ref, tf_ref, tm_ref):
    cond = cond_ref[...]
    w1 = W1_ref[...]
    w2 = W2_ref[...]
    h = jax.lax.dot_general(cond, w1, (((1,), (1,)), ((), ())),
                            preferred_element_type=jnp.float32)
    h = jnp.maximum(h + b1_ref[...], 0.0)
    bias = jax.lax.dot_general(h, w2, (((1,), (1,)), ((), ())),
                               preferred_element_type=jnp.float32)
    bias = bias + b2_ref[...]

    ctxw = ctx_ref[...]
    cf = ctxw[None, :, :] + bias[:, None, :]
    cf_ref[...] = cf

    ctxcond = jax.lax.dot_general(ctxw, cond, (((1,), (1,)), ((), ())),
                                  preferred_element_type=jnp.float32)
    ctxbias = jax.lax.dot_general(ctxw, bias, (((1,), (1,)), ((), ())),
                                  preferred_element_type=jnp.float32)
    bc = jnp.sum(bias * cond, axis=-1, keepdims=True).T
    bb = jnp.sum(bias * bias, axis=-1, keepdims=True).T
    nctx2 = jnp.sum(ctxw * ctxw, axis=-1, keepdims=True)
    num_t = ctxcond + bc
    n2_t = nctx2 + 2.0 * ctxbias + bb
    n2_t = jnp.maximum(n2_t, 0.0)
    st = num_t / jnp.maximum(jnp.sqrt(n2_t), 1e-6)

    n_io = jax.lax.broadcasted_iota(jnp.int32, (N_CTX, BB), 0)
    rank_t = jnp.zeros((N_CTX, BB), jnp.float32)
    for m in range(N_CTX):
        smrow = st[m:m + 1, :]
        beats_m = (smrow > st) | ((smrow == st) & (m < n_io))
        rank_t = rank_t + beats_m.astype(jnp.float32)
    rank = rank_t.T
    masked = jax.lax.broadcast_in_dim(rank, (BB, N_CTX, 1), (0, 1)) < float(K)

    mt = mt_ref[...]
    cm_ref[...] = jnp.where(masked, mt[None, :, :], cf)

    ctx_sum = jnp.sum(ctx_ref[...], axis=0, keepdims=True)
    tf_ref[...] = (1.0 / N_CTX) * ctx_sum + bias
    keep = (rank >= float(K)).astype(jnp.float32)
    kctx = jax.lax.dot_general(keep, ctx_ref[...], (((1,), (0,)), ((), ())),
                               preferred_element_type=jnp.float32)
    tm_ref[...] = (1.0 / N_CTX) * (kctx + float(N_CTX - K) * bias
                                   + float(K) * mt)


@functools.partial(jax.jit, static_argnames=())
def kernel(pids, cond_feat, ctx, W1, b1, W2, b2, mask_token):
    del pids
    b1r = b1.reshape(1, CTX_DIM)
    b2r = b2.reshape(1, CTX_DIM)
    mtr = mask_token.reshape(1, CTX_DIM)

    grid = (B // BB,)
    const2 = lambda i: (0, 0)
    out = pl.pallas_call(
        _fused_body,
        grid=grid,
        in_specs=[
            pl.BlockSpec((BB, COND_DIM), lambda i: (i, 0)),
            pl.BlockSpec((N_CTX, CTX_DIM), const2),
            pl.BlockSpec((CTX_DIM, COND_DIM), const2),
            pl.BlockSpec((1, CTX_DIM), const2),
            pl.BlockSpec((CTX_DIM, CTX_DIM), const2),
            pl.BlockSpec((1, CTX_DIM), const2),
            pl.BlockSpec((1, CTX_DIM), const2),
        ],
        out_specs=[
            pl.BlockSpec((BB, N_CTX, CTX_DIM), lambda i: (i, 0, 0)),
            pl.BlockSpec((BB, N_CTX, CTX_DIM), lambda i: (i, 0, 0)),
            pl.BlockSpec((BB, CTX_DIM), lambda i: (i, 0)),
            pl.BlockSpec((BB, CTX_DIM), lambda i: (i, 0)),
        ],
        out_shape=[
            jax.ShapeDtypeStruct((B, N_CTX, CTX_DIM), jnp.float32),
            jax.ShapeDtypeStruct((B, N_CTX, CTX_DIM), jnp.float32),
            jax.ShapeDtypeStruct((B, CTX_DIM), jnp.float32),
            jax.ShapeDtypeStruct((B, CTX_DIM), jnp.float32),
        ],
    )(cond_feat, ctx, W1, b1r, W2, b2r, mtr)
    ctx_full, ctx_masked, txt_full, txt_masked = out
    return (ctx_full, ctx_masked, txt_full, txt_masked)
